# Initial kernel scaffold; baseline (speedup 1.0000x reference)
import jax
import jax.numpy as jnp
from jax.experimental import pallas as pl


def kernel(x, gate_w, w1, b1, w2, b2):
    raise NotImplementedError("write your pallas kernel here")


# speedup vs baseline: 1.4344x; 1.4344x over previous
import functools

import jax
import jax.numpy as jnp
from jax.experimental import pallas as pl

D = 4096
E = 64
TOPK = 8
N_TOK = 8192
H = D // 4

BT = 256


def _router_body(x_ref, gate_w_ref, w1_ref, b1_ref, w2_ref, b2_ref,
                 logits_ref, probs_ref, tki_ref, tkp_ref, cs_ref):
    x = x_ref[...]

    logits = jax.lax.dot_general(
        x, gate_w_ref[...],
        dimension_numbers=(((1,), (1,)), ((), ())),
        preferred_element_type=jnp.float32)
    logits_ref[...] = logits

    m = jnp.max(logits, axis=-1, keepdims=True)
    ex = jnp.exp(logits - m)
    probs = ex / jnp.sum(ex, axis=-1, keepdims=True)
    probs_ref[...] = probs

    iota = jax.lax.broadcasted_iota(jnp.int32, (BT, E), 1)
    cur = probs
    for k in range(TOPK):
        mk = jnp.max(cur, axis=-1, keepdims=True)
        idx = jnp.min(jnp.where(cur == mk, iota, E), axis=-1, keepdims=True)
        tkp_ref[:, k] = mk[:, 0]
        tki_ref[:, k] = idx[:, 0]
        cur = jnp.where(iota == idx, -1.0, cur)

    h = jax.lax.dot_general(
        x, w1_ref[...],
        dimension_numbers=(((1,), (1,)), ((), ())),
        preferred_element_type=jnp.float32)
    h = jax.nn.relu(h + b1_ref[...])
    s = jnp.sum(h * w2_ref[...], axis=-1, keepdims=True)
    cs_ref[...] = jax.nn.sigmoid(s + b2_ref[0, 0])


@jax.jit
def _router(x, gate_w, w1, b1, w2, b2):
    grid = (N_TOK // BT,)
    out = pl.pallas_call(
        _router_body,
        grid=grid,
        in_specs=[
            pl.BlockSpec((BT, D), lambda i: (i, 0)),
            pl.BlockSpec((E, D), lambda i: (0, 0)),
            pl.BlockSpec((H, D), lambda i: (0, 0)),
            pl.BlockSpec((1, H), lambda i: (0, 0)),
            pl.BlockSpec((1, H), lambda i: (0, 0)),
            pl.BlockSpec((1, 1), lambda i: (0, 0)),
        ],
        out_specs=[
            pl.BlockSpec((BT, E), lambda i: (i, 0)),
            pl.BlockSpec((BT, E), lambda i: (i, 0)),
            pl.BlockSpec((BT, TOPK), lambda i: (i, 0)),
            pl.BlockSpec((BT, TOPK), lambda i: (i, 0)),
            pl.BlockSpec((BT, 1), lambda i: (i, 0)),
        ],
        out_shape=[
            jax.ShapeDtypeStruct((N_TOK, E), jnp.float32),
            jax.ShapeDtypeStruct((N_TOK, E), jnp.float32),
            jax.ShapeDtypeStruct((N_TOK, TOPK), jnp.int32),
            jax.ShapeDtypeStruct((N_TOK, TOPK), jnp.float32),
            jax.ShapeDtypeStruct((N_TOK, 1), jnp.float32),
        ],
    )(x, gate_w, w1, b1.reshape(1, H), w2, b2.reshape(1, 1))
    return out


def kernel(x, gate_w, w1, b1, w2, b2):
    logits, probs, tki, tkp, cs = _router(x, gate_w, w1, b1, w2, b2)
    capacity_factors = jnp.full((E,), 1.25, dtype=jnp.float32)
    expert_utilization = jnp.zeros((E,), dtype=jnp.float32)
    return (logits, probs, tki, tkp, cs.reshape(N_TOK),
            capacity_factors, expert_utilization)
